# baseline (speedup 1.0000x reference)
import jax
import jax.numpy as jnp
from jax import lax
from jax.experimental import pallas as pl
from jax.experimental.pallas import tpu as pltpu
from jax.experimental.pallas import tpu_sc as plsc

_NC, _NS, _L = 2, 16, 16
_KP = 512
_HCHUNK = 64


def _make_sc_loss(B, C, H, W):
    assert B == _NC * _NS and C == 2 and W & (W - 1) == 0
    w_shift = (W - 1).bit_length()
    n_chunks = H // _HCHUNK
    n_kchunks = _KP // _L
    rows_per_b = C * H
    mesh = plsc.VectorSubcoreMesh(core_axis_name="c", subcore_axis_name="s")

    def body(o2d, ind_p, twh_p, mask_p, out, idx_v, twh_v, mask_v,
             buf0, buf1, outb_v, sem0, sem1):
        b = lax.axis_index("s") * _NC + lax.axis_index("c")
        rbase = b * rows_per_b

        pltpu.sync_copy(ind_p.at[b], idx_v)
        pltpu.sync_copy(twh_p.at[b], twh_v)
        pltpu.sync_copy(mask_p.at[b], mask_v)

        bufs, sems = (buf0, buf1), (sem0, sem1)

        def issue(g):
            bf, sm = bufs[g % 2], sems[g % 2]
            h0 = _HCHUNK * g
            c0 = pltpu.async_copy(
                o2d.at[pl.ds(rbase + h0, _HCHUNK)],
                bf.at[pl.ds(0, _HCHUNK)], sm)
            c1 = pltpu.async_copy(
                o2d.at[pl.ds(rbase + H + h0, _HCHUNK)],
                bf.at[pl.ds(_HCHUNK, _HCHUNK)], sm)
            return c0, c1

        pend = issue(0)
        acc = jnp.zeros((_L,), jnp.float32)
        smv = jnp.zeros((_L,), jnp.float32)
        for g in range(n_chunks):
            nxt = issue(g + 1) if g + 1 < n_chunks else None
            for cp in pend:
                cp.wait()
            bf = bufs[g % 2]
            for j in range(n_kchunks):
                p = idx_v[pl.ds(_L * j, _L)]
                m = mask_v[pl.ds(_L * j, _L)]
                h = p >> w_shift
                w = p & (W - 1)
                rloc = h - _HCHUNK * g
                inr = (rloc >= 0) & (rloc < _HCHUNK)
                rc = jnp.minimum(jnp.maximum(rloc, 0), _HCHUNK - 1)
                wj = w >> 1
                sh = (w & 1) << 4
                g0 = plsc.load_gather(bf, [rc, wj])
                g1 = plsc.load_gather(bf, [rc + _HCHUNK, wj])
                v0 = plsc.bitcast((g0 >> sh) << 16, jnp.float32)
                v1 = plsc.bitcast((g1 >> sh) << 16, jnp.float32)
                t0 = twh_v[pl.ds(_L * j, _L)]
                t1 = twh_v[pl.ds(_KP + _L * j, _L)]
                mm = jnp.where(inr, m, jnp.float32(0.0))
                acc = acc + jnp.abs((v0 - t0) * mm) + jnp.abs((v1 - t1) * mm)
                if g == 0:
                    smv = smv + m
            pend = nxt

        outb_v[0, pl.ds(0, _L)] = acc
        outb_v[1, pl.ds(0, _L)] = smv
        pltpu.sync_copy(outb_v, out.at[b])

    return pl.kernel(
        body,
        mesh=mesh,
        out_type=jax.ShapeDtypeStruct((B, 2, _L), jnp.float32),
        scratch_types=[
            pltpu.VMEM((_KP,), jnp.int32),
            pltpu.VMEM((C * _KP,), jnp.float32),
            pltpu.VMEM((_KP,), jnp.float32),
            pltpu.VMEM((C * _HCHUNK, W // 2), jnp.int32),
            pltpu.VMEM((C * _HCHUNK, W // 2), jnp.int32),
            pltpu.VMEM((2, _L), jnp.float32),
            pltpu.SemaphoreType.DMA,
            pltpu.SemaphoreType.DMA,
        ],
        compiler_params=pltpu.CompilerParams(needs_layout_passes=False),
    )


def kernel(o_wh, t_mask, t_ind, t_wh):
    B, C, H, W = o_wh.shape
    K = t_ind.shape[1]
    o2d = jax.lax.bitcast_convert_type(
        o_wh.astype(jnp.bfloat16).reshape(B * C * H, W // 2, 2), jnp.int32)
    ind_p = jnp.pad(t_ind.astype(jnp.int32), ((0, 0), (0, _KP - K)))
    mask_p = jnp.pad(t_mask, ((0, 0), (0, _KP - K)))
    twh_p = jnp.pad(
        jnp.transpose(t_wh, (0, 2, 1)), ((0, 0), (0, 0), (0, _KP - K))
    ).reshape(B, C * _KP)
    out = _make_sc_loss(B, C, H, W)(o2d, ind_p, twh_p, mask_p)
    return out[:, 0, :].sum() / out[:, 1, :].sum()


# speedup vs baseline: 3.4150x; 3.4150x over previous
import jax
import jax.numpy as jnp
from jax import lax
from jax.experimental import pallas as pl
from jax.experimental.pallas import tpu as pltpu
from jax.experimental.pallas import tpu_sc as plsc

_NC, _NS, _L = 2, 16, 16
_KP = 512
_CHUNKS = (16, 16, 32, 64, 64, 64)
_HMAX = max(_CHUNKS)


def _make_sc_loss(B, C, H, W):
    assert B == _NC * _NS and C == 2 and W & (W - 1) == 0
    assert sum(_CHUNKS) == H
    w_shift = (W - 1).bit_length()
    n_kchunks = _KP // _L
    rows_per_b = C * H
    starts = [sum(_CHUNKS[:i]) for i in range(len(_CHUNKS))]
    mesh = plsc.VectorSubcoreMesh(core_axis_name="c", subcore_axis_name="s")

    def body(o2d, aux_p, out, aux_v, buf0, buf1, outb_v, sem0, sem1):
        b = lax.axis_index("s") * _NC + lax.axis_index("c")
        rbase = b * rows_per_b

        pltpu.sync_copy(aux_p.at[b], aux_v)

        bufs, sems = (buf0, buf1), (sem0, sem1)

        def issue(g):
            bf, sm = bufs[g % 2], sems[g % 2]
            h0, hsz = starts[g], _CHUNKS[g]
            c0 = pltpu.async_copy(
                o2d.at[pl.ds(rbase + h0, hsz)],
                bf.at[pl.ds(0, hsz)], sm)
            c1 = pltpu.async_copy(
                o2d.at[pl.ds(rbase + H + h0, hsz)],
                bf.at[pl.ds(_HMAX, hsz)], sm)
            return c0, c1

        pend = issue(0)
        acc = jnp.zeros((_L,), jnp.float32)
        smv = jnp.zeros((_L,), jnp.float32)
        for g in range(len(_CHUNKS)):
            nxt = issue(g + 1) if g + 1 < len(_CHUNKS) else None
            for cp in pend:
                cp.wait()
            bf = bufs[g % 2]
            h0, hsz = starts[g], _CHUNKS[g]
            for j in range(n_kchunks):
                p = plsc.bitcast(aux_v[pl.ds(_L * j, _L)], jnp.int32)
                m = aux_v[pl.ds(_KP + C * _KP + _L * j, _L)]
                h = p >> w_shift
                w = p & (W - 1)
                rloc = h - h0
                inr = (rloc >= 0) & (rloc < hsz)
                rc = jnp.minimum(jnp.maximum(rloc, 0), hsz - 1)
                v0 = plsc.load_gather(bf, [rc, w])
                v1 = plsc.load_gather(bf, [rc + _HMAX, w])
                t0 = aux_v[pl.ds(_KP + _L * j, _L)]
                t1 = aux_v[pl.ds(_KP + _KP + _L * j, _L)]
                mm = jnp.where(inr, m, jnp.float32(0.0))
                acc = acc + jnp.abs((v0 - t0) * mm) + jnp.abs((v1 - t1) * mm)
                if g == 0:
                    smv = smv + m
            pend = nxt

        outb_v[0, pl.ds(0, _L)] = acc
        outb_v[1, pl.ds(0, _L)] = smv
        pltpu.sync_copy(outb_v, out.at[b])

    return pl.kernel(
        body,
        mesh=mesh,
        out_type=jax.ShapeDtypeStruct((B, 2, _L), jnp.float32),
        scratch_types=[
            pltpu.VMEM(((C + 2) * _KP,), jnp.float32),
            pltpu.VMEM((C * _HMAX, W), jnp.float32),
            pltpu.VMEM((C * _HMAX, W), jnp.float32),
            pltpu.VMEM((2, _L), jnp.float32),
            pltpu.SemaphoreType.DMA,
            pltpu.SemaphoreType.DMA,
        ],
        compiler_params=pltpu.CompilerParams(needs_layout_passes=False),
    )


def kernel(o_wh, t_mask, t_ind, t_wh):
    B, C, H, W = o_wh.shape
    K = t_ind.shape[1]
    o2d = o_wh.reshape(B * C * H, W)
    ind_p = jnp.pad(t_ind.astype(jnp.int32), ((0, 0), (0, _KP - K)))
    mask_p = jnp.pad(t_mask, ((0, 0), (0, _KP - K)))
    twh_p = jnp.pad(
        jnp.transpose(t_wh, (0, 2, 1)), ((0, 0), (0, 0), (0, _KP - K))
    ).reshape(B, C * _KP)
    aux_p = jnp.concatenate(
        [jax.lax.bitcast_convert_type(ind_p, jnp.float32), twh_p, mask_p],
        axis=1)
    out = _make_sc_loss(B, C, H, W)(o2d, aux_p)
    return out[:, 0, :].sum() / out[:, 1, :].sum()


# speedup vs baseline: 3.6456x; 1.0675x over previous
import jax
import jax.numpy as jnp
from jax import lax
from jax.experimental import pallas as pl
from jax.experimental.pallas import tpu as pltpu
from jax.experimental.pallas import tpu_sc as plsc

_NC, _NS, _L = 2, 16, 16
_KP = 512
_CHUNKS = (64, 64, 64, 64)
_HMAX = max(_CHUNKS)


def _make_sc_loss(B, C, H, W):
    assert B == _NC * _NS and C == 2 and W & (W - 1) == 0
    assert sum(_CHUNKS) == H
    w_shift = (W - 1).bit_length()
    n_kchunks = _KP // _L
    rows_per_b = C * H
    starts = [sum(_CHUNKS[:i]) for i in range(len(_CHUNKS))]
    mesh = plsc.VectorSubcoreMesh(core_axis_name="c", subcore_axis_name="s")

    def body(o2d, aux_p, out, aux_v, buf0, buf1, outb_v, sem0, sem1):
        b = lax.axis_index("s") * _NC + lax.axis_index("c")
        rbase = b * rows_per_b

        pltpu.sync_copy(aux_p.at[b], aux_v)

        bufs, sems = (buf0, buf1), (sem0, sem1)

        def issue(g):
            bf, sm = bufs[g % 2], sems[g % 2]
            h0, hsz = starts[g], _CHUNKS[g]
            c0 = pltpu.async_copy(
                o2d.at[pl.ds(rbase + h0, hsz)],
                bf.at[pl.ds(0, hsz)], sm)
            c1 = pltpu.async_copy(
                o2d.at[pl.ds(rbase + H + h0, hsz)],
                bf.at[pl.ds(_HMAX, hsz)], sm)
            return c0, c1

        pend = issue(0)
        acc = jnp.zeros((_L,), jnp.float32)
        smv = jnp.zeros((_L,), jnp.float32)
        for g in range(len(_CHUNKS)):
            nxt = issue(g + 1) if g + 1 < len(_CHUNKS) else None
            for cp in pend:
                cp.wait()
            bf = bufs[g % 2]
            h0, hsz = starts[g], _CHUNKS[g]
            for j in range(n_kchunks):
                p = plsc.bitcast(aux_v[pl.ds(_L * j, _L)], jnp.int32)
                m = aux_v[pl.ds(_KP + C * _KP + _L * j, _L)]
                h = p >> w_shift
                w = p & (W - 1)
                rloc = h - h0
                inr = (rloc >= 0) & (rloc < hsz)
                rc = jnp.minimum(jnp.maximum(rloc, 0), hsz - 1)
                v0 = plsc.load_gather(bf, [rc, w])
                v1 = plsc.load_gather(bf, [rc + _HMAX, w])
                t0 = aux_v[pl.ds(_KP + _L * j, _L)]
                t1 = aux_v[pl.ds(_KP + _KP + _L * j, _L)]
                mm = jnp.where(inr, m, jnp.float32(0.0))
                acc = acc + jnp.abs((v0 - t0) * mm) + jnp.abs((v1 - t1) * mm)
                if g == 0:
                    smv = smv + m
            pend = nxt

        outb_v[0, pl.ds(0, _L)] = acc
        outb_v[1, pl.ds(0, _L)] = smv
        pltpu.sync_copy(outb_v, out.at[b])

    return pl.kernel(
        body,
        mesh=mesh,
        out_type=jax.ShapeDtypeStruct((B, 2, _L), jnp.float32),
        scratch_types=[
            pltpu.VMEM(((C + 2) * _KP,), jnp.float32),
            pltpu.VMEM((C * _HMAX, W), jnp.float32),
            pltpu.VMEM((C * _HMAX, W), jnp.float32),
            pltpu.VMEM((2, _L), jnp.float32),
            pltpu.SemaphoreType.DMA,
            pltpu.SemaphoreType.DMA,
        ],
        compiler_params=pltpu.CompilerParams(needs_layout_passes=False),
    )


def kernel(o_wh, t_mask, t_ind, t_wh):
    B, C, H, W = o_wh.shape
    K = t_ind.shape[1]
    o2d = o_wh.reshape(B * C * H, W)
    ind_p = jnp.pad(t_ind.astype(jnp.int32), ((0, 0), (0, _KP - K)))
    mask_p = jnp.pad(t_mask, ((0, 0), (0, _KP - K)))
    twh_p = jnp.pad(
        jnp.transpose(t_wh, (0, 2, 1)), ((0, 0), (0, 0), (0, _KP - K))
    ).reshape(B, C * _KP)
    aux_p = jnp.concatenate(
        [jax.lax.bitcast_convert_type(ind_p, jnp.float32), twh_p, mask_p],
        axis=1)
    out = _make_sc_loss(B, C, H, W)(o2d, aux_p)
    return out[:, 0, :].sum() / out[:, 1, :].sum()


# speedup vs baseline: 3.6459x; 1.0001x over previous
import jax
import jax.numpy as jnp
from jax import lax
from jax.experimental import pallas as pl
from jax.experimental.pallas import tpu as pltpu
from jax.experimental.pallas import tpu_sc as plsc

_NC, _NS, _L = 2, 16, 16
_KP = 512
_CHUNKS = (32, 64, 80, 80)
_HMAX = max(_CHUNKS)


def _make_sc_loss(B, C, H, W):
    assert B == _NC * _NS and C == 2 and W & (W - 1) == 0
    assert sum(_CHUNKS) == H
    w_shift = (W - 1).bit_length()
    n_kchunks = _KP // _L
    rows_per_b = C * H
    starts = [sum(_CHUNKS[:i]) for i in range(len(_CHUNKS))]
    mesh = plsc.VectorSubcoreMesh(core_axis_name="c", subcore_axis_name="s")

    def body(o2d, aux_p, out, aux_v, buf0, buf1, outb_v, sem0, sem1):
        b = lax.axis_index("s") * _NC + lax.axis_index("c")
        rbase = b * rows_per_b

        pltpu.sync_copy(aux_p.at[b], aux_v)

        bufs, sems = (buf0, buf1), (sem0, sem1)

        def issue(g):
            bf, sm = bufs[g % 2], sems[g % 2]
            h0, hsz = starts[g], _CHUNKS[g]
            c0 = pltpu.async_copy(
                o2d.at[pl.ds(rbase + h0, hsz)],
                bf.at[pl.ds(0, hsz)], sm)
            c1 = pltpu.async_copy(
                o2d.at[pl.ds(rbase + H + h0, hsz)],
                bf.at[pl.ds(_HMAX, hsz)], sm)
            return c0, c1

        pend = issue(0)
        acc = jnp.zeros((_L,), jnp.float32)
        smv = jnp.zeros((_L,), jnp.float32)
        for g in range(len(_CHUNKS)):
            nxt = issue(g + 1) if g + 1 < len(_CHUNKS) else None
            for cp in pend:
                cp.wait()
            bf = bufs[g % 2]
            h0, hsz = starts[g], _CHUNKS[g]
            for j in range(n_kchunks):
                p = plsc.bitcast(aux_v[pl.ds(_L * j, _L)], jnp.int32)
                m = aux_v[pl.ds(_KP + C * _KP + _L * j, _L)]
                h = p >> w_shift
                w = p & (W - 1)
                rloc = h - h0
                inr = (rloc >= 0) & (rloc < hsz)
                rc = jnp.minimum(jnp.maximum(rloc, 0), hsz - 1)
                v0 = plsc.load_gather(bf, [rc, w])
                v1 = plsc.load_gather(bf, [rc + _HMAX, w])
                t0 = aux_v[pl.ds(_KP + _L * j, _L)]
                t1 = aux_v[pl.ds(_KP + _KP + _L * j, _L)]
                mm = jnp.where(inr, m, jnp.float32(0.0))
                acc = acc + jnp.abs((v0 - t0) * mm) + jnp.abs((v1 - t1) * mm)
                if g == 0:
                    smv = smv + m
            pend = nxt

        outb_v[0, pl.ds(0, _L)] = acc
        outb_v[1, pl.ds(0, _L)] = smv
        pltpu.sync_copy(outb_v, out.at[b])

    return pl.kernel(
        body,
        mesh=mesh,
        out_type=jax.ShapeDtypeStruct((B, 2, _L), jnp.float32),
        scratch_types=[
            pltpu.VMEM(((C + 2) * _KP,), jnp.float32),
            pltpu.VMEM((C * _HMAX, W), jnp.float32),
            pltpu.VMEM((C * _HMAX, W), jnp.float32),
            pltpu.VMEM((2, _L), jnp.float32),
            pltpu.SemaphoreType.DMA,
            pltpu.SemaphoreType.DMA,
        ],
        compiler_params=pltpu.CompilerParams(needs_layout_passes=False),
    )


def kernel(o_wh, t_mask, t_ind, t_wh):
    B, C, H, W = o_wh.shape
    K = t_ind.shape[1]
    o2d = o_wh.reshape(B * C * H, W)
    ind_p = jnp.pad(t_ind.astype(jnp.int32), ((0, 0), (0, _KP - K)))
    mask_p = jnp.pad(t_mask, ((0, 0), (0, _KP - K)))
    twh_p = jnp.pad(
        jnp.transpose(t_wh, (0, 2, 1)), ((0, 0), (0, 0), (0, _KP - K))
    ).reshape(B, C * _KP)
    aux_p = jnp.concatenate(
        [jax.lax.bitcast_convert_type(ind_p, jnp.float32), twh_p, mask_p],
        axis=1)
    out = _make_sc_loss(B, C, H, W)(o2d, aux_p)
    return out[:, 0, :].sum() / out[:, 1, :].sum()


# speedup vs baseline: 3.6603x; 1.0040x over previous
import jax
import jax.numpy as jnp
from jax import lax
from jax.experimental import pallas as pl
from jax.experimental.pallas import tpu as pltpu
from jax.experimental.pallas import tpu_sc as plsc

_NC, _NS, _L = 2, 16, 16
_KP = 512
_CHUNKS = (64, 64, 64, 64)
_HMAX = max(_CHUNKS)


def _make_sc_loss(B, C, H, W):
    assert B == _NC * _NS and C == 2 and W & (W - 1) == 0
    assert sum(_CHUNKS) == H
    w_shift = (W - 1).bit_length()
    n_kchunks = _KP // _L
    rows_per_b = C * H
    starts = [sum(_CHUNKS[:i]) for i in range(len(_CHUNKS))]
    mesh = plsc.VectorSubcoreMesh(core_axis_name="c", subcore_axis_name="s")

    def body(o2d, aux_p, out, aux_v, buf0, buf1, outb_v, sem0, sem1):
        b = lax.axis_index("s") * _NC + lax.axis_index("c")
        rbase = b * rows_per_b

        pltpu.sync_copy(aux_p.at[b], aux_v)

        bufs, sems = (buf0, buf1), (sem0, sem1)

        def issue(g):
            bf, sm = bufs[g % 2], sems[g % 2]
            h0, hsz = starts[g], _CHUNKS[g]
            c0 = pltpu.async_copy(
                o2d.at[pl.ds(rbase + h0, hsz)],
                bf.at[pl.ds(0, hsz)], sm)
            c1 = pltpu.async_copy(
                o2d.at[pl.ds(rbase + H + h0, hsz)],
                bf.at[pl.ds(_HMAX, hsz)], sm)
            return c0, c1

        pend = issue(0)
        acc = jnp.zeros((_L,), jnp.float32)
        smv = jnp.zeros((_L,), jnp.float32)
        for g in range(len(_CHUNKS)):
            nxt = issue(g + 1) if g + 1 < len(_CHUNKS) else None
            for cp in pend:
                cp.wait()
            bf = bufs[g % 2]
            h0, hsz = starts[g], _CHUNKS[g]
            for j in range(n_kchunks):
                p = plsc.bitcast(aux_v[pl.ds(_L * j, _L)], jnp.int32)
                m = aux_v[pl.ds(_KP + C * _KP + _L * j, _L)]
                h = p >> w_shift
                w = p & (W - 1)
                rloc = h - h0
                inr = (rloc >= 0) & (rloc < hsz)
                rc = jnp.minimum(jnp.maximum(rloc, 0), hsz - 1)
                v0 = plsc.load_gather(bf, [rc, w])
                v1 = plsc.load_gather(bf, [rc + _HMAX, w])
                t0 = aux_v[pl.ds(_KP + _L * j, _L)]
                t1 = aux_v[pl.ds(_KP + _KP + _L * j, _L)]
                mm = jnp.where(inr, m, jnp.float32(0.0))
                acc = acc + jnp.abs((v0 - t0) * mm) + jnp.abs((v1 - t1) * mm)
                if g == 0:
                    smv = smv + m
            pend = nxt

        outb_v[0, pl.ds(0, _L)] = acc
        outb_v[1, pl.ds(0, _L)] = smv
        pltpu.sync_copy(outb_v, out.at[b])

    return pl.kernel(
        body,
        mesh=mesh,
        out_type=jax.ShapeDtypeStruct((B, 2, _L), jnp.float32),
        scratch_types=[
            pltpu.VMEM(((C + 2) * _KP,), jnp.float32),
            pltpu.VMEM((C * _HMAX, W), jnp.float32),
            pltpu.VMEM((C * _HMAX, W), jnp.float32),
            pltpu.VMEM((2, _L), jnp.float32),
            pltpu.SemaphoreType.DMA,
            pltpu.SemaphoreType.DMA,
        ],
        compiler_params=pltpu.CompilerParams(needs_layout_passes=False),
    )


def kernel(o_wh, t_mask, t_ind, t_wh):
    B, C, H, W = o_wh.shape
    K = t_ind.shape[1]
    o2d = o_wh.reshape(B * C * H, W)
    ind_p = jnp.pad(t_ind.astype(jnp.int32), ((0, 0), (0, _KP - K)))
    mask_p = jnp.pad(t_mask, ((0, 0), (0, _KP - K)))
    twh_p = jnp.pad(
        jnp.transpose(t_wh, (0, 2, 1)), ((0, 0), (0, 0), (0, _KP - K))
    ).reshape(B, C * _KP)
    aux_p = jnp.concatenate(
        [jax.lax.bitcast_convert_type(ind_p, jnp.float32), twh_p, mask_p],
        axis=1)
    out = _make_sc_loss(B, C, H, W)(o2d, aux_p)
    return out[:, 0, :].sum() / out[:, 1, :].sum()


# speedup vs baseline: 3.7748x; 1.0313x over previous
import jax
import jax.numpy as jnp
from jax import lax
from jax.experimental import pallas as pl
from jax.experimental.pallas import tpu as pltpu
from jax.experimental.pallas import tpu_sc as plsc

_NC, _NS, _L = 2, 16, 16
_KP = 512
_CHUNKS = (64, 96, 96)
_HMAX = max(_CHUNKS)


def _make_sc_loss(B, C, H, W):
    assert B == _NC * _NS and C == 2 and W & (W - 1) == 0
    assert sum(_CHUNKS) == H
    w_shift = (W - 1).bit_length()
    n_kchunks = _KP // _L
    rows_per_b = C * H
    starts = [sum(_CHUNKS[:i]) for i in range(len(_CHUNKS))]
    mesh = plsc.VectorSubcoreMesh(core_axis_name="c", subcore_axis_name="s")

    def body(o2d, aux_p, out, aux_v, buf0, buf1, outb_v, sem0, sem1):
        b = lax.axis_index("s") * _NC + lax.axis_index("c")
        rbase = b * rows_per_b

        pltpu.sync_copy(aux_p.at[b], aux_v)

        bufs, sems = (buf0, buf1), (sem0, sem1)

        def issue(g):
            bf, sm = bufs[g % 2], sems[g % 2]
            h0, hsz = starts[g], _CHUNKS[g]
            c0 = pltpu.async_copy(
                o2d.at[pl.ds(rbase + h0, hsz)],
                bf.at[pl.ds(0, hsz)], sm)
            c1 = pltpu.async_copy(
                o2d.at[pl.ds(rbase + H + h0, hsz)],
                bf.at[pl.ds(_HMAX, hsz)], sm)
            return c0, c1

        pend = issue(0)
        acc = jnp.zeros((_L,), jnp.float32)
        smv = jnp.zeros((_L,), jnp.float32)
        for g in range(len(_CHUNKS)):
            nxt = issue(g + 1) if g + 1 < len(_CHUNKS) else None
            for cp in pend:
                cp.wait()
            bf = bufs[g % 2]
            h0, hsz = starts[g], _CHUNKS[g]
            for j in range(n_kchunks):
                p = plsc.bitcast(aux_v[pl.ds(_L * j, _L)], jnp.int32)
                m = aux_v[pl.ds(_KP + C * _KP + _L * j, _L)]
                h = p >> w_shift
                w = p & (W - 1)
                rloc = h - h0
                inr = (rloc >= 0) & (rloc < hsz)
                rc = jnp.minimum(jnp.maximum(rloc, 0), hsz - 1)
                v0 = plsc.load_gather(bf, [rc, w])
                v1 = plsc.load_gather(bf, [rc + _HMAX, w])
                t0 = aux_v[pl.ds(_KP + _L * j, _L)]
                t1 = aux_v[pl.ds(_KP + _KP + _L * j, _L)]
                mm = jnp.where(inr, m, jnp.float32(0.0))
                acc = acc + jnp.abs((v0 - t0) * mm) + jnp.abs((v1 - t1) * mm)
                if g == 0:
                    smv = smv + m
            pend = nxt

        outb_v[0, pl.ds(0, _L)] = acc
        outb_v[1, pl.ds(0, _L)] = smv
        pltpu.sync_copy(outb_v, out.at[b])

    return pl.kernel(
        body,
        mesh=mesh,
        out_type=jax.ShapeDtypeStruct((B, 2, _L), jnp.float32),
        scratch_types=[
            pltpu.VMEM(((C + 2) * _KP,), jnp.float32),
            pltpu.VMEM((C * _HMAX, W), jnp.float32),
            pltpu.VMEM((C * _HMAX, W), jnp.float32),
            pltpu.VMEM((2, _L), jnp.float32),
            pltpu.SemaphoreType.DMA,
            pltpu.SemaphoreType.DMA,
        ],
        compiler_params=pltpu.CompilerParams(needs_layout_passes=False),
    )


def kernel(o_wh, t_mask, t_ind, t_wh):
    B, C, H, W = o_wh.shape
    K = t_ind.shape[1]
    o2d = o_wh.reshape(B * C * H, W)
    ind_p = jnp.pad(t_ind.astype(jnp.int32), ((0, 0), (0, _KP - K)))
    mask_p = jnp.pad(t_mask, ((0, 0), (0, _KP - K)))
    twh_p = jnp.pad(
        jnp.transpose(t_wh, (0, 2, 1)), ((0, 0), (0, 0), (0, _KP - K))
    ).reshape(B, C * _KP)
    aux_p = jnp.concatenate(
        [jax.lax.bitcast_convert_type(ind_p, jnp.float32), twh_p, mask_p],
        axis=1)
    out = _make_sc_loss(B, C, H, W)(o2d, aux_p)
    return out[:, 0, :].sum() / out[:, 1, :].sum()


# speedup vs baseline: 3.8405x; 1.0174x over previous
import jax
import jax.numpy as jnp
from jax import lax
from jax.experimental import pallas as pl
from jax.experimental.pallas import tpu as pltpu
from jax.experimental.pallas import tpu_sc as plsc

_NC, _NS, _L = 2, 16, 16
_KP = 512
_CHUNKS = (32, 112, 112)
_HMAX = max(_CHUNKS)


def _make_sc_loss(B, C, H, W):
    assert B == _NC * _NS and C == 2 and W & (W - 1) == 0
    assert sum(_CHUNKS) == H
    w_shift = (W - 1).bit_length()
    n_kchunks = _KP // _L
    rows_per_b = C * H
    starts = [sum(_CHUNKS[:i]) for i in range(len(_CHUNKS))]
    mesh = plsc.VectorSubcoreMesh(core_axis_name="c", subcore_axis_name="s")

    def body(o2d, aux_p, out, aux_v, buf0, buf1, outb_v, sem0, sem1):
        b = lax.axis_index("s") * _NC + lax.axis_index("c")
        rbase = b * rows_per_b

        pltpu.sync_copy(aux_p.at[b], aux_v)

        bufs, sems = (buf0, buf1), (sem0, sem1)

        def issue(g):
            bf, sm = bufs[g % 2], sems[g % 2]
            h0, hsz = starts[g], _CHUNKS[g]
            c0 = pltpu.async_copy(
                o2d.at[pl.ds(rbase + h0, hsz)],
                bf.at[pl.ds(0, hsz)], sm)
            c1 = pltpu.async_copy(
                o2d.at[pl.ds(rbase + H + h0, hsz)],
                bf.at[pl.ds(_HMAX, hsz)], sm)
            return c0, c1

        pend = issue(0)
        acc = jnp.zeros((_L,), jnp.float32)
        smv = jnp.zeros((_L,), jnp.float32)
        for g in range(len(_CHUNKS)):
            nxt = issue(g + 1) if g + 1 < len(_CHUNKS) else None
            for cp in pend:
                cp.wait()
            bf = bufs[g % 2]
            h0, hsz = starts[g], _CHUNKS[g]
            for j in range(n_kchunks):
                p = plsc.bitcast(aux_v[pl.ds(_L * j, _L)], jnp.int32)
                m = aux_v[pl.ds(_KP + C * _KP + _L * j, _L)]
                h = p >> w_shift
                w = p & (W - 1)
                rloc = h - h0
                inr = (rloc >= 0) & (rloc < hsz)
                rc = jnp.minimum(jnp.maximum(rloc, 0), hsz - 1)
                v0 = plsc.load_gather(bf, [rc, w])
                v1 = plsc.load_gather(bf, [rc + _HMAX, w])
                t0 = aux_v[pl.ds(_KP + _L * j, _L)]
                t1 = aux_v[pl.ds(_KP + _KP + _L * j, _L)]
                mm = jnp.where(inr, m, jnp.float32(0.0))
                acc = acc + jnp.abs((v0 - t0) * mm) + jnp.abs((v1 - t1) * mm)
                if g == 0:
                    smv = smv + m
            pend = nxt

        outb_v[0, pl.ds(0, _L)] = acc
        outb_v[1, pl.ds(0, _L)] = smv
        pltpu.sync_copy(outb_v, out.at[b])

    return pl.kernel(
        body,
        mesh=mesh,
        out_type=jax.ShapeDtypeStruct((B, 2, _L), jnp.float32),
        scratch_types=[
            pltpu.VMEM(((C + 2) * _KP,), jnp.float32),
            pltpu.VMEM((C * _HMAX, W), jnp.float32),
            pltpu.VMEM((C * _HMAX, W), jnp.float32),
            pltpu.VMEM((2, _L), jnp.float32),
            pltpu.SemaphoreType.DMA,
            pltpu.SemaphoreType.DMA,
        ],
        compiler_params=pltpu.CompilerParams(needs_layout_passes=False),
    )


def kernel(o_wh, t_mask, t_ind, t_wh):
    B, C, H, W = o_wh.shape
    K = t_ind.shape[1]
    o2d = o_wh.reshape(B * C * H, W)
    ind_p = jnp.pad(t_ind.astype(jnp.int32), ((0, 0), (0, _KP - K)))
    mask_p = jnp.pad(t_mask, ((0, 0), (0, _KP - K)))
    twh_p = jnp.pad(
        jnp.transpose(t_wh, (0, 2, 1)), ((0, 0), (0, 0), (0, _KP - K))
    ).reshape(B, C * _KP)
    aux_p = jnp.concatenate(
        [jax.lax.bitcast_convert_type(ind_p, jnp.float32), twh_p, mask_p],
        axis=1)
    out = _make_sc_loss(B, C, H, W)(o2d, aux_p)
    return out[:, 0, :].sum() / out[:, 1, :].sum()
